# Initial kernel scaffold; baseline (speedup 1.0000x reference)
import jax
import jax.numpy as jnp
from jax.experimental import pallas as pl


def kernel(x, edge_index, edge_attr, Wl0, bl0, Wr0, br0, att0, We0, cb0, lg0, lb0, Wl1, bl1, Wr1, br1, att1, We1, cb1, lg1, lb1, Wl2, bl2, Wr2, br2, att2, We2, cb2, lg2, lb2, Wp, bp):
    raise NotImplementedError("write your pallas kernel here")


# speedup vs baseline: 5.8876x; 5.8876x over previous
import functools

import jax
import jax.numpy as jnp
from jax import lax
from jax.experimental import pallas as pl
from jax.experimental.pallas import tpu as pltpu
from jax.experimental.pallas import tpu_sc as plsc

N = 10000
D = 128
H = 8
C = 16
HC = H * C
ED = 4
E = 320000
E_REAL = E + N
E_PAD = 344064
NW = 32
NPW = 320
NPAD = NW * NPW
K1 = 768
CH1 = E_PAD // NW // K1
TB = 4096
B3 = 8192
NB3 = E_PAD // B3
G3 = 112
EID_BITS = 19
RB = 2000
NEG = -1e30


def _pre_body(h_ref, wl_ref, bl_ref, wr_ref, br_ref, xl_ref, xr_ref):
    h = h_ref[...]
    xl_ref[...] = jnp.dot(h, wl_ref[...], preferred_element_type=jnp.float32) + bl_ref[...]
    xr_ref[...] = jnp.dot(h, wr_ref[...], preferred_element_type=jnp.float32) + br_ref[...]


def _tc_pre(h, Wl, bl, Wr, br):
    return pl.pallas_call(
        _pre_body,
        grid=(N // RB,),
        in_specs=[
            pl.BlockSpec((RB, D), lambda i: (i, 0)),
            pl.BlockSpec((D, HC), lambda i: (0, 0)),
            pl.BlockSpec((1, HC), lambda i: (0, 0)),
            pl.BlockSpec((D, HC), lambda i: (0, 0)),
            pl.BlockSpec((1, HC), lambda i: (0, 0)),
        ],
        out_specs=[pl.BlockSpec((RB, HC), lambda i: (i, 0)),
                   pl.BlockSpec((RB, HC), lambda i: (i, 0))],
        out_shape=[jax.ShapeDtypeStruct((N, HC), jnp.float32),
                   jax.ShapeDtypeStruct((N, HC), jnp.float32)],
    )(h, Wl, bl.reshape(1, HC), Wr, br.reshape(1, HC))


def _sc_gather(xl, xr, src, dst):
    mesh = plsc.VectorSubcoreMesh(
        core_axis_name="c", subcore_axis_name="s", num_cores=2, num_subcores=16)

    @functools.partial(
        pl.kernel,
        out_type=[jax.ShapeDtypeStruct((E_PAD, D), jnp.float32),
                  jax.ShapeDtypeStruct((E_PAD, D), jnp.float32)],
        mesh=mesh,
        compiler_params=pltpu.CompilerParams(needs_layout_passes=False),
        scratch_types=[
            pltpu.VMEM((K1,), jnp.int32),
            pltpu.VMEM((K1, D), jnp.float32),
            pltpu.SemaphoreType.DMA,
        ],
    )
    def k(xl_hbm, xr_hbm, src_hbm, dst_hbm, gl_hbm, gr_hbm, idx_v, rows_v, sem):
        wid = lax.axis_index("s") * 2 + lax.axis_index("c")
        base = wid * (E_PAD // NW)

        def chunk(j, carry):
            off = base + j * K1
            pltpu.sync_copy(src_hbm.at[pl.ds(off, K1)], idx_v)
            pltpu.async_copy(xl_hbm.at[idx_v], rows_v, sem).wait()
            pltpu.sync_copy(rows_v, gl_hbm.at[pl.ds(off, K1)])
            pltpu.sync_copy(dst_hbm.at[pl.ds(off, K1)], idx_v)
            pltpu.async_copy(xr_hbm.at[idx_v], rows_v, sem).wait()
            pltpu.sync_copy(rows_v, gr_hbm.at[pl.ds(off, K1)])
            return carry

        lax.fori_loop(0, CH1, chunk, 0)

    return k(xl, xr, src, dst)


def _alpha_body(gl_ref, gr_ref, ea_ref, we_ref, att_ref, alpha_ref, gmax_ref):
    i = pl.program_id(0)
    em = jnp.dot(ea_ref[...], we_ref[...], preferred_element_type=jnp.float32)
    m = gl_ref[...] + gr_ref[...] + em
    m = jnp.where(m >= 0, m, 0.2 * m)
    am = m * att_ref[...]
    a = jnp.sum(am.reshape(TB, H, C), axis=2)
    rid = i * TB + lax.broadcasted_iota(jnp.int32, (TB, 1), 0)
    a = jnp.where(rid < E_REAL, a, NEG)
    alpha_ref[...] = a
    bm = jnp.max(a, axis=0)
    bmt = jnp.concatenate([bm, bm]).reshape(1, 2 * H)

    @pl.when(i == 0)
    def _():
        gmax_ref[...] = jnp.full((1, 2 * H), NEG, jnp.float32)

    gmax_ref[...] = jnp.maximum(gmax_ref[...], bmt)


def _tc_alpha(gl, gr, ea, We, att):
    return pl.pallas_call(
        _alpha_body,
        grid=(E_PAD // TB,),
        in_specs=[
            pl.BlockSpec((TB, D), lambda i: (i, 0)),
            pl.BlockSpec((TB, D), lambda i: (i, 0)),
            pl.BlockSpec((TB, ED), lambda i: (i, 0)),
            pl.BlockSpec((ED, HC), lambda i: (0, 0)),
            pl.BlockSpec((1, HC), lambda i: (0, 0)),
        ],
        out_specs=[pl.BlockSpec((TB, H), lambda i: (i, 0)),
                   pl.BlockSpec((1, 2 * H), lambda i: (0, 0))],
        out_shape=[jax.ShapeDtypeStruct((E_PAD, H), jnp.float32),
                   jax.ShapeDtypeStruct((1, 2 * H), jnp.float32)],
    )(gl, gr, ea, We, att.reshape(1, HC))


def _sc_accum(comb, gmaxt, dst):
    mesh = plsc.VectorSubcoreMesh(
        core_axis_name="c", subcore_axis_name="s", num_cores=2, num_subcores=16)

    @functools.partial(
        pl.kernel,
        out_type=[jax.ShapeDtypeStruct((NPAD, D), jnp.float32),
                  jax.ShapeDtypeStruct((NPAD, 2 * H), jnp.float32)],
        mesh=mesh,
        compiler_params=pltpu.CompilerParams(needs_layout_passes=False),
        scratch_types=[
            pltpu.VMEM((NPW + 1, D), jnp.float32),
            pltpu.VMEM((NPW + 1, 2 * H), jnp.float32),
            pltpu.VMEM((B3,), jnp.int32),
            pltpu.VMEM((B3 + G3 + 16,), jnp.int32),
            pltpu.VMEM((2 * G3,), jnp.int32),
            pltpu.VMEM((2 * G3, D), jnp.float32),
            pltpu.VMEM((1, 2 * H), jnp.float32),
            pltpu.SemaphoreType.DMA,
        ],
    )
    def k(comb_hbm, gmax_hbm, dst_hbm, outp_hbm, denp_hbm,
          acc_out, acc_den, dst_buf, eid_buf, gidx_buf, comb_g, gm_v, sem):
        wid = lax.axis_index("s") * 2 + lax.axis_index("c")
        lo = wid * NPW
        z16 = jnp.zeros((16,), jnp.float32)
        pad16 = jnp.full((16,), NPW << EID_BITS, jnp.int32)
        iota16 = lax.iota(jnp.int32, 16)
        lane8 = iota16 & 7
        emask = (1 << EID_BITS) - 1

        def zacc(r, carry):
            for hh in range(H):
                acc_out[r, pl.ds(hh * 16, 16)] = z16
            acc_den[r, pl.ds(0, 16)] = z16
            return carry

        lax.fori_loop(0, NPW + 1, zacc, 0)

        pltpu.sync_copy(gmax_hbm, gm_v)
        gmvec = gm_v[0, pl.ds(0, 16)]

        def block(j, carry):
            pltpu.sync_copy(dst_hbm.at[pl.ds(j * B3, B3)], dst_buf)

            def compact(i, c):
                v = dst_buf[pl.ds(i * 16, 16)]
                m = (v >= lo) & (v < lo + NPW)
                eidv = (j * B3 + i * 16) + iota16
                ent = eidv | ((v - lo) << EID_BITS)
                cs = plsc.cumsum(m.astype(jnp.int32))
                pos = c + cs - 1
                plsc.store_scatter(eid_buf, [pos], ent, mask=m)
                return c + cs[15]

            c = lax.fori_loop(0, B3 // 16, compact, 0, unroll=4)

            for t in range(G3 // 16):
                eid_buf[pl.ds(c + t * 16, 16)] = pad16

            nchunks = (c + G3 - 1) // G3

            def chunk(kk, carry2):
                base = kk * G3
                for t in range(G3 // 16):
                    ent = eid_buf[pl.ds(base + t * 16, 16)]
                    eidv = ent & emask
                    gidx_buf[pl.ds(t * 16, 16)] = eidv
                    gidx_buf[pl.ds(G3 + t * 16, 16)] = (
                        lax.shift_right_logical(eidv, 4) + E_PAD)
                pltpu.async_copy(comb_hbm.at[gidx_buf], comb_g, sem).wait()

                def accum(e, cc):
                    ent = eid_buf[pl.ds(base + e, 16)][0]
                    nl = lax.shift_right_logical(ent, EID_BITS)
                    coloff = (ent & 15) * 8
                    av = plsc.load_gather(
                        comb_g, [jnp.full((16,), G3 + e, jnp.int32), coloff + lane8])
                    ex2 = jnp.exp(av - gmvec)
                    plsc.addupdate(acc_den.at[nl, pl.ds(0, 16)], ex2)
                    for hh in range(H):
                        evec = ex2.at[jnp.full((16,), hh, jnp.int32)].get(
                            mode="promise_in_bounds")
                        g = comb_g[e, pl.ds(hh * 16, 16)]
                        plsc.addupdate(acc_out.at[nl, pl.ds(hh * 16, 16)], evec * g)
                    return cc

                lax.fori_loop(0, G3, accum, 0, unroll=2)
                return carry2

            lax.fori_loop(0, nchunks, chunk, 0)
            return carry

        lax.fori_loop(0, NB3, block, 0)

        pltpu.sync_copy(acc_out.at[pl.ds(0, NPW)], outp_hbm.at[pl.ds(lo, NPW)])
        pltpu.sync_copy(acc_den.at[pl.ds(0, NPW)], denp_hbm.at[pl.ds(lo, NPW)])

    return k(comb, gmaxt, dst)


def _post_body(o_ref, d_ref, cb_ref, lg_ref, lb_ref, hp_ref, out_ref):
    o = o_ref[...].reshape(RB, H, C)
    den = d_ref[...][:, 0:H]
    o = o / (den[:, :, None] + 1e-16)
    o = o.reshape(RB, HC) + cb_ref[...]
    o = jnp.where(o > 0, o, jnp.exp(o) - 1.0)
    mu = jnp.mean(o, axis=1, keepdims=True)
    xc = o - mu
    var = jnp.mean(xc * xc, axis=1, keepdims=True)
    y = xc * lax.rsqrt(var + 1e-5) * lg_ref[...] + lb_ref[...]
    out_ref[...] = y + hp_ref[...]


def _tc_post(outp, denp, cb, lg, lb, hprev):
    return pl.pallas_call(
        _post_body,
        grid=(N // RB,),
        in_specs=[
            pl.BlockSpec((RB, D), lambda i: (i, 0)),
            pl.BlockSpec((RB, 2 * H), lambda i: (i, 0)),
            pl.BlockSpec((1, HC), lambda i: (0, 0)),
            pl.BlockSpec((1, HC), lambda i: (0, 0)),
            pl.BlockSpec((1, HC), lambda i: (0, 0)),
            pl.BlockSpec((RB, D), lambda i: (i, 0)),
        ],
        out_specs=pl.BlockSpec((RB, D), lambda i: (i, 0)),
        out_shape=jax.ShapeDtypeStruct((N, D), jnp.float32),
    )(outp, denp, cb.reshape(1, HC), lg.reshape(1, HC), lb.reshape(1, HC), hprev)


def _post2_body(o_ref, d_ref, cb_ref, lg_ref, lb_ref, wp_ref, bp_ref, out_ref):
    o = o_ref[...].reshape(RB, H, C)
    den = d_ref[...][:, 0:H]
    o = o / (den[:, :, None] + 1e-16)
    o = jnp.mean(o, axis=1)
    o = o + cb_ref[...]
    o = jnp.where(o > 0, o, jnp.exp(o) - 1.0)
    mu = jnp.mean(o, axis=1, keepdims=True)
    xc = o - mu
    var = jnp.mean(xc * xc, axis=1, keepdims=True)
    y = xc * lax.rsqrt(var + 1e-5) * lg_ref[...] + lb_ref[...]
    out_ref[...] = jnp.dot(y, wp_ref[...], preferred_element_type=jnp.float32) + bp_ref[...]


def _tc_post2(outp, denp, cb, lg, lb, Wp, bp):
    return pl.pallas_call(
        _post2_body,
        grid=(N // RB,),
        in_specs=[
            pl.BlockSpec((RB, D), lambda i: (i, 0)),
            pl.BlockSpec((RB, 2 * H), lambda i: (i, 0)),
            pl.BlockSpec((1, C), lambda i: (0, 0)),
            pl.BlockSpec((1, C), lambda i: (0, 0)),
            pl.BlockSpec((1, C), lambda i: (0, 0)),
            pl.BlockSpec((C, D), lambda i: (0, 0)),
            pl.BlockSpec((1, D), lambda i: (0, 0)),
        ],
        out_specs=pl.BlockSpec((RB, D), lambda i: (i, 0)),
        out_shape=jax.ShapeDtypeStruct((N, D), jnp.float32),
    )(outp, denp, cb.reshape(1, C), lg.reshape(1, C), lb.reshape(1, C),
      Wp, bp.reshape(1, D))


def kernel(x, edge_index, edge_attr,
           Wl0, bl0, Wr0, br0, att0, We0, cb0, lg0, lb0,
           Wl1, bl1, Wr1, br1, att1, We1, cb1, lg1, lb1,
           Wl2, bl2, Wr2, br2, att2, We2, cb2, lg2, lb2,
           Wp, bp):
    loop = jnp.arange(N, dtype=jnp.int32)
    padlen = E_PAD - E_REAL
    zpad = jnp.zeros((padlen,), jnp.int32)
    src = jnp.concatenate([edge_index[0].astype(jnp.int32), loop, zpad])
    dst = jnp.concatenate([edge_index[1].astype(jnp.int32), loop, zpad])
    fill = jnp.mean(edge_attr, axis=0, keepdims=True)
    ea = jnp.concatenate(
        [edge_attr, jnp.tile(fill, (N, 1)), jnp.zeros((padlen, ED), jnp.float32)],
        axis=0)

    params = [(Wl0, bl0, Wr0, br0, att0, We0, cb0, lg0, lb0),
              (Wl1, bl1, Wr1, br1, att1, We1, cb1, lg1, lb1),
              (Wl2, bl2, Wr2, br2, att2, We2, cb2, lg2, lb2)]
    h = x
    for l in range(3):
        Wl, bl, Wr, br, att, We, cb, lg, lb = params[l]
        xl, xr = _tc_pre(h, Wl, bl, Wr, br)
        gl, gr = _sc_gather(xl, xr, src, dst)
        alpha, gmaxt = _tc_alpha(gl, gr, ea, We, att)
        comb = jnp.concatenate([gl, alpha.reshape(E_PAD // 16, D)], axis=0)
        outp, denp = _sc_accum(comb, gmaxt, dst)
        if l < 2:
            h = _tc_post(outp[:N], denp[:N], cb, lg, lb, h)
        else:
            h = _tc_post2(outp[:N], denp[:N], cb, lg, lb, Wp, bp)
    return h


# speedup vs baseline: 5.8939x; 1.0011x over previous
import functools

import jax
import jax.numpy as jnp
from jax import lax
from jax.experimental import pallas as pl
from jax.experimental.pallas import tpu as pltpu
from jax.experimental.pallas import tpu_sc as plsc

N = 10000
D = 128
H = 8
C = 16
HC = H * C
ED = 4
E = 320000
E_REAL = E + N
E_PAD = 344064
NW = 32
NPW = 320
NPAD = NW * NPW
K1 = 768
CH1 = E_PAD // NW // K1
TB = 4096
B3 = 8192
NB3 = E_PAD // B3
G3 = 112
EID_BITS = 19
RB = 2000
NEG = -1e30


def _pre_body(h_ref, wl_ref, bl_ref, wr_ref, br_ref, xl_ref, xr_ref):
    h = h_ref[...]
    xl_ref[...] = jnp.dot(h, wl_ref[...], preferred_element_type=jnp.float32) + bl_ref[...]
    xr_ref[...] = jnp.dot(h, wr_ref[...], preferred_element_type=jnp.float32) + br_ref[...]


def _tc_pre(h, Wl, bl, Wr, br):
    return pl.pallas_call(
        _pre_body,
        grid=(N // RB,),
        in_specs=[
            pl.BlockSpec((RB, D), lambda i: (i, 0)),
            pl.BlockSpec((D, HC), lambda i: (0, 0)),
            pl.BlockSpec((1, HC), lambda i: (0, 0)),
            pl.BlockSpec((D, HC), lambda i: (0, 0)),
            pl.BlockSpec((1, HC), lambda i: (0, 0)),
        ],
        out_specs=[pl.BlockSpec((RB, HC), lambda i: (i, 0)),
                   pl.BlockSpec((RB, HC), lambda i: (i, 0))],
        out_shape=[jax.ShapeDtypeStruct((N, HC), jnp.float32),
                   jax.ShapeDtypeStruct((N, HC), jnp.float32)],
    )(h, Wl, bl.reshape(1, HC), Wr, br.reshape(1, HC))


def _sc_gather(xl, xr, src, dst):
    mesh = plsc.VectorSubcoreMesh(
        core_axis_name="c", subcore_axis_name="s", num_cores=2, num_subcores=16)

    @functools.partial(
        pl.kernel,
        out_type=[jax.ShapeDtypeStruct((E_PAD, D), jnp.float32),
                  jax.ShapeDtypeStruct((E_PAD, D), jnp.float32)],
        mesh=mesh,
        compiler_params=pltpu.CompilerParams(needs_layout_passes=False),
        scratch_types=[
            pltpu.VMEM((K1,), jnp.int32),
            pltpu.VMEM((K1, D), jnp.float32),
            pltpu.SemaphoreType.DMA,
        ],
    )
    def k(xl_hbm, xr_hbm, src_hbm, dst_hbm, gl_hbm, gr_hbm, idx_v, rows_v, sem):
        wid = lax.axis_index("s") * 2 + lax.axis_index("c")
        base = wid * (E_PAD // NW)

        def chunk(j, carry):
            off = base + j * K1
            pltpu.sync_copy(src_hbm.at[pl.ds(off, K1)], idx_v)
            pltpu.async_copy(xl_hbm.at[idx_v], rows_v, sem).wait()
            pltpu.sync_copy(rows_v, gl_hbm.at[pl.ds(off, K1)])
            pltpu.sync_copy(dst_hbm.at[pl.ds(off, K1)], idx_v)
            pltpu.async_copy(xr_hbm.at[idx_v], rows_v, sem).wait()
            pltpu.sync_copy(rows_v, gr_hbm.at[pl.ds(off, K1)])
            return carry

        lax.fori_loop(0, CH1, chunk, 0)

    return k(xl, xr, src, dst)


def _alpha_body(gl_ref, gr_ref, ea_ref, we_ref, att_ref, alpha_ref, gmax_ref):
    i = pl.program_id(0)
    em = jnp.dot(ea_ref[...], we_ref[...], preferred_element_type=jnp.float32)
    m = gl_ref[...] + gr_ref[...] + em
    m = jnp.where(m >= 0, m, 0.2 * m)
    am = m * att_ref[...]
    a = jnp.sum(am.reshape(TB, H, C), axis=2)
    rid = i * TB + lax.broadcasted_iota(jnp.int32, (TB, 1), 0)
    a = jnp.where(rid < E_REAL, a, NEG)
    alpha_ref[...] = a
    bm = jnp.max(a, axis=0)
    bmt = jnp.concatenate([bm, bm]).reshape(1, 2 * H)

    @pl.when(i == 0)
    def _():
        gmax_ref[...] = jnp.full((1, 2 * H), NEG, jnp.float32)

    gmax_ref[...] = jnp.maximum(gmax_ref[...], bmt)


def _tc_alpha(gl, gr, ea, We, att):
    return pl.pallas_call(
        _alpha_body,
        grid=(E_PAD // TB,),
        in_specs=[
            pl.BlockSpec((TB, D), lambda i: (i, 0)),
            pl.BlockSpec((TB, D), lambda i: (i, 0)),
            pl.BlockSpec((TB, ED), lambda i: (i, 0)),
            pl.BlockSpec((ED, HC), lambda i: (0, 0)),
            pl.BlockSpec((1, HC), lambda i: (0, 0)),
        ],
        out_specs=[pl.BlockSpec((TB, H), lambda i: (i, 0)),
                   pl.BlockSpec((1, 2 * H), lambda i: (0, 0))],
        out_shape=[jax.ShapeDtypeStruct((E_PAD, H), jnp.float32),
                   jax.ShapeDtypeStruct((1, 2 * H), jnp.float32)],
    )(gl, gr, ea, We, att.reshape(1, HC))


def _sc_accum(comb, gmaxt, dst):
    mesh = plsc.VectorSubcoreMesh(
        core_axis_name="c", subcore_axis_name="s", num_cores=2, num_subcores=16)

    @functools.partial(
        pl.kernel,
        out_type=[jax.ShapeDtypeStruct((NPAD, D), jnp.float32),
                  jax.ShapeDtypeStruct((NPAD, 2 * H), jnp.float32)],
        mesh=mesh,
        compiler_params=pltpu.CompilerParams(needs_layout_passes=False),
        scratch_types=[
            pltpu.VMEM((NPW + 1, D), jnp.float32),
            pltpu.VMEM((NPW + 1, 2 * H), jnp.float32),
            pltpu.VMEM((B3,), jnp.int32),
            pltpu.VMEM((B3 + G3 + 16,), jnp.int32),
            pltpu.VMEM((2 * G3,), jnp.int32),
            pltpu.VMEM((2 * G3, D), jnp.float32),
            pltpu.VMEM((1, 2 * H), jnp.float32),
            pltpu.SemaphoreType.DMA,
            pltpu.SemaphoreType.DMA,
        ],
    )
    def k(comb_hbm, gmax_hbm, dst_hbm, outp_hbm, denp_hbm,
          acc_out, acc_den, dst_buf, eid_buf, gidx_buf, comb_g, gm_v, sem,
          sem_d):
        wid = lax.axis_index("s") * 2 + lax.axis_index("c")
        lo = wid * NPW
        z16 = jnp.zeros((16,), jnp.float32)
        pad16 = jnp.full((16,), NPW << EID_BITS, jnp.int32)
        iota16 = lax.iota(jnp.int32, 16)
        lane8 = iota16 & 7
        emask = (1 << EID_BITS) - 1

        def zacc(r, carry):
            for hh in range(H):
                acc_out[r, pl.ds(hh * 16, 16)] = z16
            acc_den[r, pl.ds(0, 16)] = z16
            return carry

        lax.fori_loop(0, NPW + 1, zacc, 0)

        pltpu.sync_copy(gmax_hbm, gm_v)
        gmvec = gm_v[0, pl.ds(0, 16)]

        pltpu.async_copy(dst_hbm.at[pl.ds(0, B3)], dst_buf, sem_d)

        def block(j, carry):
            pltpu.make_async_copy(
                dst_hbm.at[pl.ds(j * B3, B3)], dst_buf, sem_d).wait()

            def compact(i, c):
                v = dst_buf[pl.ds(i * 16, 16)]
                m = (v >= lo) & (v < lo + NPW)
                eidv = (j * B3 + i * 16) + iota16
                ent = eidv | ((v - lo) << EID_BITS)
                cs = plsc.cumsum(m.astype(jnp.int32))
                pos = c + cs - 1
                plsc.store_scatter(eid_buf, [pos], ent, mask=m)
                return c + cs[15]

            c = lax.fori_loop(0, B3 // 16, compact, 0, unroll=4)

            @pl.when(j < NB3 - 1)
            def _():
                pltpu.async_copy(
                    dst_hbm.at[pl.ds((j + 1) * B3, B3)], dst_buf, sem_d)

            for t in range(G3 // 16):
                eid_buf[pl.ds(c + t * 16, 16)] = pad16

            nchunks = (c + G3 - 1) // G3

            def chunk(kk, carry2):
                base = kk * G3
                for t in range(G3 // 16):
                    ent = eid_buf[pl.ds(base + t * 16, 16)]
                    eidv = ent & emask
                    gidx_buf[pl.ds(t * 16, 16)] = eidv
                    gidx_buf[pl.ds(G3 + t * 16, 16)] = (
                        lax.shift_right_logical(eidv, 4) + E_PAD)
                pltpu.async_copy(comb_hbm.at[gidx_buf], comb_g, sem).wait()

                def accum(e, cc):
                    ent = eid_buf[pl.ds(base + e, 16)][0]
                    nl = lax.shift_right_logical(ent, EID_BITS)
                    coloff = (ent & 15) * 8
                    av = plsc.load_gather(
                        comb_g, [jnp.full((16,), G3 + e, jnp.int32), coloff + lane8])
                    ex2 = jnp.exp(av - gmvec)
                    plsc.addupdate(acc_den.at[nl, pl.ds(0, 16)], ex2)
                    for hh in range(H):
                        evec = ex2.at[jnp.full((16,), hh, jnp.int32)].get(
                            mode="promise_in_bounds")
                        g = comb_g[e, pl.ds(hh * 16, 16)]
                        plsc.addupdate(acc_out.at[nl, pl.ds(hh * 16, 16)], evec * g)
                    return cc

                lax.fori_loop(0, G3, accum, 0, unroll=2)
                return carry2

            lax.fori_loop(0, nchunks, chunk, 0)
            return carry

        lax.fori_loop(0, NB3, block, 0)

        pltpu.sync_copy(acc_out.at[pl.ds(0, NPW)], outp_hbm.at[pl.ds(lo, NPW)])
        pltpu.sync_copy(acc_den.at[pl.ds(0, NPW)], denp_hbm.at[pl.ds(lo, NPW)])

    return k(comb, gmaxt, dst)


def _post_body(o_ref, d_ref, cb_ref, lg_ref, lb_ref, hp_ref, out_ref):
    o = o_ref[...].reshape(RB, H, C)
    den = d_ref[...][:, 0:H]
    o = o / (den[:, :, None] + 1e-16)
    o = o.reshape(RB, HC) + cb_ref[...]
    o = jnp.where(o > 0, o, jnp.exp(o) - 1.0)
    mu = jnp.mean(o, axis=1, keepdims=True)
    xc = o - mu
    var = jnp.mean(xc * xc, axis=1, keepdims=True)
    y = xc * lax.rsqrt(var + 1e-5) * lg_ref[...] + lb_ref[...]
    out_ref[...] = y + hp_ref[...]


def _tc_post(outp, denp, cb, lg, lb, hprev):
    return pl.pallas_call(
        _post_body,
        grid=(N // RB,),
        in_specs=[
            pl.BlockSpec((RB, D), lambda i: (i, 0)),
            pl.BlockSpec((RB, 2 * H), lambda i: (i, 0)),
            pl.BlockSpec((1, HC), lambda i: (0, 0)),
            pl.BlockSpec((1, HC), lambda i: (0, 0)),
            pl.BlockSpec((1, HC), lambda i: (0, 0)),
            pl.BlockSpec((RB, D), lambda i: (i, 0)),
        ],
        out_specs=pl.BlockSpec((RB, D), lambda i: (i, 0)),
        out_shape=jax.ShapeDtypeStruct((N, D), jnp.float32),
    )(outp, denp, cb.reshape(1, HC), lg.reshape(1, HC), lb.reshape(1, HC), hprev)


def _post2_body(o_ref, d_ref, cb_ref, lg_ref, lb_ref, wp_ref, bp_ref, out_ref):
    o = o_ref[...].reshape(RB, H, C)
    den = d_ref[...][:, 0:H]
    o = o / (den[:, :, None] + 1e-16)
    o = jnp.mean(o, axis=1)
    o = o + cb_ref[...]
    o = jnp.where(o > 0, o, jnp.exp(o) - 1.0)
    mu = jnp.mean(o, axis=1, keepdims=True)
    xc = o - mu
    var = jnp.mean(xc * xc, axis=1, keepdims=True)
    y = xc * lax.rsqrt(var + 1e-5) * lg_ref[...] + lb_ref[...]
    out_ref[...] = jnp.dot(y, wp_ref[...], preferred_element_type=jnp.float32) + bp_ref[...]


def _tc_post2(outp, denp, cb, lg, lb, Wp, bp):
    return pl.pallas_call(
        _post2_body,
        grid=(N // RB,),
        in_specs=[
            pl.BlockSpec((RB, D), lambda i: (i, 0)),
            pl.BlockSpec((RB, 2 * H), lambda i: (i, 0)),
            pl.BlockSpec((1, C), lambda i: (0, 0)),
            pl.BlockSpec((1, C), lambda i: (0, 0)),
            pl.BlockSpec((1, C), lambda i: (0, 0)),
            pl.BlockSpec((C, D), lambda i: (0, 0)),
            pl.BlockSpec((1, D), lambda i: (0, 0)),
        ],
        out_specs=pl.BlockSpec((RB, D), lambda i: (i, 0)),
        out_shape=jax.ShapeDtypeStruct((N, D), jnp.float32),
    )(outp, denp, cb.reshape(1, C), lg.reshape(1, C), lb.reshape(1, C),
      Wp, bp.reshape(1, D))


def kernel(x, edge_index, edge_attr,
           Wl0, bl0, Wr0, br0, att0, We0, cb0, lg0, lb0,
           Wl1, bl1, Wr1, br1, att1, We1, cb1, lg1, lb1,
           Wl2, bl2, Wr2, br2, att2, We2, cb2, lg2, lb2,
           Wp, bp):
    loop = jnp.arange(N, dtype=jnp.int32)
    padlen = E_PAD - E_REAL
    zpad = jnp.zeros((padlen,), jnp.int32)
    src = jnp.concatenate([edge_index[0].astype(jnp.int32), loop, zpad])
    dst = jnp.concatenate([edge_index[1].astype(jnp.int32), loop, zpad])
    fill = jnp.mean(edge_attr, axis=0, keepdims=True)
    ea = jnp.concatenate(
        [edge_attr, jnp.tile(fill, (N, 1)), jnp.zeros((padlen, ED), jnp.float32)],
        axis=0)

    params = [(Wl0, bl0, Wr0, br0, att0, We0, cb0, lg0, lb0),
              (Wl1, bl1, Wr1, br1, att1, We1, cb1, lg1, lb1),
              (Wl2, bl2, Wr2, br2, att2, We2, cb2, lg2, lb2)]
    h = x
    for l in range(3):
        Wl, bl, Wr, br, att, We, cb, lg, lb = params[l]
        xl, xr = _tc_pre(h, Wl, bl, Wr, br)
        gl, gr = _sc_gather(xl, xr, src, dst)
        alpha, gmaxt = _tc_alpha(gl, gr, ea, We, att)
        comb = jnp.concatenate([gl, alpha.reshape(E_PAD // 16, D)], axis=0)
        outp, denp = _sc_accum(comb, gmaxt, dst)
        if l < 2:
            h = _tc_post(outp[:N], denp[:N], cb, lg, lb, h)
        else:
            h = _tc_post2(outp[:N], denp[:N], cb, lg, lb, Wp, bp)
    return h
